# Initial kernel scaffold; baseline (speedup 1.0000x reference)
import jax
import jax.numpy as jnp
from jax.experimental import pallas as pl


def kernel(x, edge_index, W1, b1, g1, be1, a1, W2, b2, g2, be2, a2):
    raise NotImplementedError("write your pallas kernel here")


# speedup vs baseline: 4.2369x; 4.2369x over previous
import functools

import jax
import jax.numpy as jnp
from jax import lax
from jax.experimental import pallas as pl
from jax.experimental.pallas import tpu as pltpu
from jax.experimental.pallas import tpu_sc as plsc

N = 10000
E = 320000
D_IN = 128
D_HID = 512

NC = 2
NS = 16
ROWS_PER_TILE = N // NS
EDGES_PER_CORE = E // NC
EDGES_PER_TILE = EDGES_PER_CORE // NS
CHUNK = 80
NCHUNK = EDGES_PER_TILE // CHUNK

_EPS = 1e-5


def _sc_agg_body(x_hbm, src_hbm, dst_hbm, zeros_hbm, out_hbm,
                 src_v, dst_v, rows_v, acc_sh, sem):
    cid = lax.axis_index("c")
    sid = lax.axis_index("s")

    pltpu.sync_copy(zeros_hbm, acc_sh.at[pl.ds(sid * ROWS_PER_TILE, ROWS_PER_TILE)])
    plsc.subcore_barrier()

    base = cid * EDGES_PER_CORE + sid * EDGES_PER_TILE

    def body(i, carry):
        off = pl.multiple_of(base + i * CHUNK, 8)
        pltpu.sync_copy(src_hbm.at[pl.ds(off, CHUNK)], src_v)
        pltpu.sync_copy(dst_hbm.at[pl.ds(off, CHUNK)], dst_v)
        pltpu.async_copy(x_hbm.at[src_v], rows_v, sem).wait()
        pltpu.sync_copy(rows_v, acc_sh.at[dst_v], add=True)
        return carry

    lax.fori_loop(0, NCHUNK, body, 0)
    plsc.subcore_barrier()

    pltpu.sync_copy(acc_sh.at[pl.ds(sid * ROWS_PER_TILE, ROWS_PER_TILE)],
                    out_hbm.at[cid * NS + sid])


@functools.cache
def _get_sc_agg():
    mesh = plsc.VectorSubcoreMesh(core_axis_name="c", subcore_axis_name="s",
                                  num_cores=NC, num_subcores=NS)
    return pl.kernel(
        _sc_agg_body,
        out_type=jax.ShapeDtypeStruct((NC * NS, ROWS_PER_TILE, D_IN), jnp.float32),
        mesh=mesh,
        scratch_types=[
            pltpu.VMEM((CHUNK,), jnp.int32),
            pltpu.VMEM((CHUNK,), jnp.int32),
            pltpu.VMEM((CHUNK, D_IN), jnp.float32),
            pltpu.VMEM_SHARED((N, D_IN), jnp.float32),
            pltpu.SemaphoreType.DMA,
        ],
    )


BLK = 1000
NB = N // BLK

_TC_PARAMS = pltpu.CompilerParams(dimension_semantics=("arbitrary",))


def _matmul_t(a, w):
    return lax.dot_general(a, w, (((1,), (1,)), ((), ())),
                           preferred_element_type=jnp.float32,
                           precision=lax.Precision.HIGHEST)


def _t1_body(x_ref, p0_ref, p1_ref, w1_ref, b1_ref, z1_ref, stats_ref):
    i = pl.program_id(0)
    h = x_ref[...] + p0_ref[...] + p1_ref[...]
    z = _matmul_t(h, w1_ref[...]) + b1_ref[...]
    z1_ref[...] = z

    @pl.when(i == 0)
    def _():
        stats_ref[...] = jnp.zeros_like(stats_ref)

    stats_ref[0:1, :] += jnp.sum(z, axis=0, keepdims=True)
    stats_ref[1:2, :] += jnp.sum(z * z, axis=0, keepdims=True)


def _bn_prelu(z, stats, g, be, a):
    mean = stats[0:1, :] * (1.0 / N)
    var = stats[1:2, :] * (1.0 / N) - mean * mean
    xhat = (z - mean) * lax.rsqrt(var + _EPS)
    o = xhat * g + be
    return jnp.maximum(o, 0.0) + a * jnp.minimum(o, 0.0)


def _t2_body(z1_ref, stats1_ref, g1_ref, be1_ref, a1_ref, w2_ref, b2_ref,
             z2_ref, stats_ref):
    i = pl.program_id(0)
    x1 = _bn_prelu(z1_ref[...], stats1_ref[...], g1_ref[...], be1_ref[...],
                   a1_ref[0, 0])
    z = _matmul_t(x1, w2_ref[...]) + b2_ref[...]
    z2_ref[...] = z

    @pl.when(i == 0)
    def _():
        stats_ref[...] = jnp.zeros_like(stats_ref)

    stats_ref[0:1, :] += jnp.sum(z, axis=0, keepdims=True)
    stats_ref[1:2, :] += jnp.sum(z * z, axis=0, keepdims=True)


def _t3_body(z2_ref, stats2_ref, g2_ref, be2_ref, a2_ref, x_ref, out_ref):
    o = _bn_prelu(z2_ref[...], stats2_ref[...], g2_ref[...], be2_ref[...],
                  a2_ref[0, 0])
    out_ref[...] = o + x_ref[...]


def _row_spec(d):
    return pl.BlockSpec((BLK, d), lambda i: (i, 0))


def _full_spec(shape):
    return pl.BlockSpec(shape, lambda i: tuple(0 for _ in shape))


_t1 = pl.pallas_call(
    _t1_body,
    grid=(NB,),
    in_specs=[_row_spec(D_IN), _row_spec(D_IN), _row_spec(D_IN),
              _full_spec((D_HID, D_IN)), _full_spec((1, D_HID))],
    out_specs=[_row_spec(D_HID), _full_spec((2, D_HID))],
    out_shape=[jax.ShapeDtypeStruct((N, D_HID), jnp.float32),
               jax.ShapeDtypeStruct((2, D_HID), jnp.float32)],
    compiler_params=_TC_PARAMS,
)

_t2 = pl.pallas_call(
    _t2_body,
    grid=(NB,),
    in_specs=[_row_spec(D_HID), _full_spec((2, D_HID)), _full_spec((1, D_HID)),
              _full_spec((1, D_HID)), _full_spec((1, 1)),
              _full_spec((D_IN, D_HID)), _full_spec((1, D_IN))],
    out_specs=[_row_spec(D_IN), _full_spec((2, D_IN))],
    out_shape=[jax.ShapeDtypeStruct((N, D_IN), jnp.float32),
               jax.ShapeDtypeStruct((2, D_IN), jnp.float32)],
    compiler_params=_TC_PARAMS,
)

_t3 = pl.pallas_call(
    _t3_body,
    grid=(NB,),
    in_specs=[_row_spec(D_IN), _full_spec((2, D_IN)), _full_spec((1, D_IN)),
              _full_spec((1, D_IN)), _full_spec((1, 1)), _row_spec(D_IN)],
    out_specs=_row_spec(D_IN),
    out_shape=jax.ShapeDtypeStruct((N, D_IN), jnp.float32),
    compiler_params=_TC_PARAMS,
)


def kernel(x, edge_index, W1, b1, g1, be1, a1, W2, b2, g2, be2, a2):
    src = edge_index[0].astype(jnp.int32)
    dst = edge_index[1].astype(jnp.int32)
    zeros = jnp.zeros((ROWS_PER_TILE, D_IN), jnp.float32)

    parts = _get_sc_agg()(x, src, dst, zeros).reshape(NC, N, D_IN)

    z1, stats1 = _t1(x, parts[0], parts[1], W1, b1.reshape(1, D_HID))
    z2, stats2 = _t2(z1, stats1, g1.reshape(1, D_HID), be1.reshape(1, D_HID),
                     a1.reshape(1, 1), W2, b2.reshape(1, D_IN))
    out = _t3(z2, stats2, g2.reshape(1, D_IN), be2.reshape(1, D_IN),
              a2.reshape(1, 1), x)
    return out
